# baseline (speedup 1.0000x reference)
import functools

import jax
import jax.numpy as jnp
from jax import lax
from jax.experimental import pallas as pl
from jax.experimental.pallas import tpu as pltpu
from jax.experimental.pallas import tpu_sc as plsc

_HIDDEN = 1024
_LANES = 16
_NCORES = 2
_NSUB = 16
_NW = _NCORES * _NSUB

_CHUNK = 8
_NBUF = 4


def _emb_body(ids_hbm, pids_hbm, word_hbm, ptab_hbm, out_hbm,
              idx_w, idx_p, *bufs_and_sems, per_w, nchunk):
    w = bufs_and_sems[0:_NBUF]
    p = bufs_and_sems[_NBUF:2 * _NBUF]
    o = bufs_and_sems[2 * _NBUF:3 * _NBUF]
    sw = bufs_and_sems[3 * _NBUF:4 * _NBUF]
    sp = bufs_and_sems[4 * _NBUF:5 * _NBUF]
    so = bufs_and_sems[5 * _NBUF:6 * _NBUF]

    wid = lax.axis_index("s") * _NCORES + lax.axis_index("c")
    base = wid * per_w
    pltpu.sync_copy(ids_hbm.at[pl.ds(base, per_w)], idx_w)
    pltpu.sync_copy(pids_hbm.at[pl.ds(base, per_w)], idx_p)

    def gather_pair(ci, b):
        off = ci * _CHUNK
        pltpu.make_async_copy(word_hbm.at[idx_w.at[pl.ds(off, _CHUNK)]],
                              w[b], sw[b]).start()
        pltpu.make_async_copy(ptab_hbm.at[idx_p.at[pl.ds(off, _CHUNK)]],
                              p[b], sp[b]).start()

    def wait_gather(ci, b):
        off = ci * _CHUNK
        pltpu.make_async_copy(word_hbm.at[idx_w.at[pl.ds(off, _CHUNK)]],
                              w[b], sw[b]).wait()
        pltpu.make_async_copy(ptab_hbm.at[idx_p.at[pl.ds(off, _CHUNK)]],
                              p[b], sp[b]).wait()

    def start_out(ci, b):
        pltpu.make_async_copy(o[b], out_hbm.at[pl.ds(base + ci * _CHUNK,
                                                     _CHUNK)], so[b]).start()

    def wait_out(ci, b):
        pltpu.make_async_copy(o[b], out_hbm.at[pl.ds(base + ci * _CHUNK,
                                                     _CHUNK)], so[b]).wait()

    for ci in range(_NBUF - 1):
        gather_pair(ci, ci)

    n_outer = nchunk // _NBUF

    def ring_body(g, carry):
        for b in range(_NBUF):
            ci = g * _NBUF + b
            nb = (b + _NBUF - 1) % _NBUF
            if b == 0:
                gather_pair(ci + _NBUF - 1, nb)
            else:
                @pl.when(g < n_outer - 1)
                def _():
                    gather_pair(ci + _NBUF - 1, nb)
            wait_gather(ci, b)

            @pl.when(g > 0)
            def _():
                wait_out(ci - _NBUF, b)

            def add_row(r, c2):
                for j in range(_HIDDEN // _LANES):
                    sl = pl.ds(j * _LANES, _LANES)
                    o[b][r, sl] = w[b][r, sl] + p[b][r, sl]
                return c2

            lax.fori_loop(0, _CHUNK, add_row, 0, unroll=False)
            start_out(ci, b)
        return carry

    lax.fori_loop(0, n_outer, ring_body, 0, unroll=False)
    for b in range(_NBUF):
        wait_out(nchunk - _NBUF + b, b)


def kernel(input_ids, position_ids, word_table, pos_table):
    b, s = input_ids.shape
    n = b * s
    per_w = n // _NW
    nchunk = per_w // _CHUNK
    ids = input_ids.reshape(n).astype(jnp.int32)
    pids = position_ids.reshape(n).astype(jnp.int32)

    mesh = plsc.VectorSubcoreMesh(core_axis_name="c", subcore_axis_name="s")
    scratch = [pltpu.VMEM((per_w,), jnp.int32),
               pltpu.VMEM((per_w,), jnp.int32)]
    scratch += [pltpu.VMEM((_CHUNK, _HIDDEN), jnp.float32)
                for _ in range(3 * _NBUF)]
    scratch += [pltpu.SemaphoreType.DMA for _ in range(3 * _NBUF)]
    grid_kernel = pl.kernel(
        functools.partial(_emb_body, per_w=per_w, nchunk=nchunk),
        mesh=mesh,
        out_type=jax.ShapeDtypeStruct((n, _HIDDEN), jnp.float32),
        scratch_types=scratch,
    )
    out = grid_kernel(ids, pids, word_table, pos_table)
    return out.reshape(b, s, _HIDDEN)
